# Initial kernel scaffold; baseline (speedup 1.0000x reference)
import jax
import jax.numpy as jnp
from jax.experimental import pallas as pl


def kernel(x, edge_index, W, a_src, a_dst, bias):
    raise NotImplementedError("write your pallas kernel here")


# speedup vs baseline: 37.6040x; 37.6040x over previous
import functools

import jax
import jax.numpy as jnp
from jax import lax
from jax.experimental import pallas as pl
from jax.experimental.pallas import tpu as pltpu
from jax.experimental.pallas import tpu_sc as plsc

N_NODES = 10000
D_IN = 128
N_HEADS = 8
D_OUT = 16
DMODEL = N_HEADS * D_OUT
NEG_SLOPE = 0.2

NP = 10112
CHUNK = 48
N_TILES = 32


def _tc1_body(x_ref, w_ref, a_ref, feat_ref, t_ref):
    feat = jnp.dot(x_ref[...], w_ref[...], preferred_element_type=jnp.float32)
    feat_ref[...] = feat
    t_ref[...] = jnp.dot(feat, a_ref[...], preferred_element_type=jnp.float32)


def _tc1(x, w, a_pack):
    return pl.pallas_call(
        _tc1_body,
        out_shape=[
            jax.ShapeDtypeStruct((N_NODES, DMODEL), jnp.float32),
            jax.ShapeDtypeStruct((N_NODES, 2 * N_HEADS), jnp.float32),
        ],
    )(x, w, a_pack)


_GATHER_DNUMS = lax.GatherDimensionNumbers(
    offset_dims=(), collapsed_slice_dims=(0,), start_index_map=(0,))


def _splat(v, idx):
    return lax.gather(v, idx[:, None], _GATHER_DNUMS, slice_sizes=(1,),
                      mode=lax.GatherScatterMode.PROMISE_IN_BOUNDS)
def _sc_edge_kernel(n_chunks_per_tile):
    mesh = plsc.VectorSubcoreMesh(core_axis_name="c", subcore_axis_name="s")
    ep_per_tile = n_chunks_per_tile * CHUNK
    rows_per_tile = NP // 16

    @functools.partial(
        pl.kernel,
        mesh=mesh,
        out_type=[
            jax.ShapeDtypeStruct((2, NP, DMODEL), jnp.float32),
            jax.ShapeDtypeStruct((2, NP, 16), jnp.float32),
        ],
        scratch_types=[
            pltpu.VMEM((CHUNK,), jnp.int32),
            pltpu.VMEM((CHUNK,), jnp.int32),
            pltpu.VMEM((CHUNK, DMODEL), jnp.float32),
            pltpu.VMEM((CHUNK, 16), jnp.float32),
            pltpu.VMEM((CHUNK, 16), jnp.float32),
            pltpu.VMEM((CHUNK, DMODEL), jnp.float32),
            pltpu.VMEM((CHUNK, 16), jnp.float32),
            pltpu.VMEM_SHARED((NP, DMODEL), jnp.float32),
            pltpu.VMEM_SHARED((NP, 16), jnp.float32),
            pltpu.SemaphoreType.DMA,
        ],
        compiler_params=pltpu.CompilerParams(use_tc_tiling_on_sc=False),
    )
    def k(feat_hbm, t1_hbm, t2_hbm, src_hbm, dst_hbm, zacc_hbm, zden_hbm,
          acc_out, den_out,
          src_v, dst_v, fs_v, ts_v, td_v, msg_v, ee_v, acc_s, den_s, sem):
        cid = lax.axis_index("c")
        sid = lax.axis_index("s")
        wid = sid * 2 + cid

        r0 = sid * rows_per_tile
        pltpu.sync_copy(zacc_hbm.at[pl.ds(r0, rows_per_tile)],
                        acc_s.at[pl.ds(r0, rows_per_tile)])
        pltpu.sync_copy(zden_hbm.at[pl.ds(r0, rows_per_tile)],
                        den_s.at[pl.ds(r0, rows_per_tile)])
        plsc.subcore_barrier()

        lane = lax.iota(jnp.int32, 16)
        head_lo = lane < 8

        def edge_body(j, _):
            a = ts_v[j, :]
            b = td_v[j, :]
            z = a + b
            e = jnp.where(z > 0, z, NEG_SLOPE * z)
            ee = jnp.where(head_lo, jnp.exp(e), 0.0)
            ee_v[j, :] = ee
            for h in range(N_HEADS):
                idx = jnp.full((16,), h, jnp.int32)
                w = _splat(ee, idx)
                f = fs_v[j, pl.ds(16 * h, 16)]
                msg_v[j, pl.ds(16 * h, 16)] = f * w
            return _

        def chunk_body(ci, _):
            off = (wid * n_chunks_per_tile + ci) * CHUNK
            pltpu.sync_copy(src_hbm.at[pl.ds(off, CHUNK)], src_v)
            pltpu.sync_copy(dst_hbm.at[pl.ds(off, CHUNK)], dst_v)
            c1 = pltpu.async_copy(feat_hbm.at[src_v], fs_v, sem)
            c2 = pltpu.async_copy(t1_hbm.at[src_v], ts_v, sem)
            c3 = pltpu.async_copy(t2_hbm.at[dst_v], td_v, sem)
            c1.wait()
            c2.wait()
            c3.wait()
            lax.fori_loop(0, CHUNK, edge_body, None)
            pltpu.sync_copy(ee_v, den_s.at[dst_v], add=True)
            pltpu.sync_copy(msg_v, acc_s.at[dst_v], add=True)
            return _

        lax.fori_loop(0, n_chunks_per_tile, chunk_body, None)
        plsc.subcore_barrier()

        pltpu.sync_copy(acc_s.at[pl.ds(r0, rows_per_tile)],
                        acc_out.at[cid, pl.ds(r0, rows_per_tile)])
        pltpu.sync_copy(den_s.at[pl.ds(r0, rows_per_tile)],
                        den_out.at[cid, pl.ds(r0, rows_per_tile)])

    return k


def _tc3_body(acc_ref, den_ref, k_ref, bias_ref, out_ref):
    a = acc_ref[0, :N_NODES, :] + acc_ref[1, :N_NODES, :]
    d = den_ref[0, :N_NODES, :] + den_ref[1, :N_NODES, :]
    d128 = jnp.dot(d, k_ref[...], preferred_element_type=jnp.float32)
    v = a / (d128 + 1e-9) + bias_ref[...][None, :]
    neg = jnp.exp(jnp.minimum(v, 0.0)) - 1.0
    out_ref[...] = jnp.where(v > 0, v, neg)


def _tc3(acc, den, k_exp, bias):
    return pl.pallas_call(
        _tc3_body,
        out_shape=jax.ShapeDtypeStruct((N_NODES, DMODEL), jnp.float32),
    )(acc, den, k_exp, bias)


def kernel(x, edge_index, W, a_src, a_dst, bias):
    n_edges = edge_index.shape[1]
    n_chunks_per_tile = -(-n_edges // (N_TILES * CHUNK))
    ep = n_chunks_per_tile * N_TILES * CHUNK

    hh = jnp.arange(DMODEL, dtype=jnp.int32) // D_OUT
    onehot = (hh[:, None] == jnp.arange(N_HEADS, dtype=jnp.int32)[None, :])
    onehot = onehot.astype(jnp.float32)
    a_pack = jnp.concatenate(
        [a_src.reshape(DMODEL)[:, None] * onehot,
         a_dst.reshape(DMODEL)[:, None] * onehot], axis=1)

    feat, t = _tc1(x, W, a_pack)

    pad_n = NP - N_NODES
    featp = jnp.pad(feat, ((0, pad_n), (0, 0)))
    tp = jnp.pad(t, ((0, pad_n), (0, 0)))
    t1 = tp
    t2 = jnp.concatenate([tp[:, 8:], tp[:, :8]], 1)

    ei = edge_index.astype(jnp.int32)
    pad_e = ep - n_edges
    src = jnp.concatenate([ei[0], jnp.full((pad_e,), NP - 1, jnp.int32)])
    dst = jnp.concatenate([ei[1], jnp.full((pad_e,), NP - 1, jnp.int32)])

    zacc = jnp.zeros((NP, DMODEL), jnp.float32)
    zden = jnp.zeros((NP, 16), jnp.float32)

    acc, den = _sc_edge_kernel(n_chunks_per_tile)(
        featp, t1, t2, src, dst, zacc, zden)

    k_exp = (jnp.arange(16, dtype=jnp.int32)[:, None]
             == jnp.arange(DMODEL, dtype=jnp.int32)[None, :] // D_OUT)
    k_exp = k_exp.astype(jnp.float32)
    return _tc3(acc, den, k_exp, bias)
